# baseline (speedup 1.0000x reference)
import functools

import jax
import jax.numpy as jnp
from jax import lax
from jax.experimental import pallas as pl
from jax.experimental.pallas import tpu as pltpu
from jax.experimental.pallas import tpu_sc as plsc

E = 8
TOP_K = 2
TM = 256
NEG_INF = -1e30

SC_CORES = 2
SC_SUBCORES = 16
NW = SC_CORES * SC_SUBCORES


def _router_body(x_ref, gw_ref, idx_ref, prob_ref):
    scores = jax.lax.dot_general(
        x_ref[...], gw_ref[...], (((1,), (1,)), ((), ())),
        preferred_element_type=jnp.float32,
        precision=jax.lax.Precision.DEFAULT)
    tr = scores.shape[0]
    iota = jax.lax.broadcasted_iota(jnp.int32, (tr, E), 1)
    m1 = jnp.max(scores, axis=1, keepdims=True)
    a1 = jnp.min(jnp.where(scores == m1, iota, E), axis=1, keepdims=True)
    masked = jnp.where(iota == a1, NEG_INF, scores)
    m2 = jnp.max(masked, axis=1, keepdims=True)
    a2 = jnp.min(jnp.where(masked == m2, iota, E), axis=1, keepdims=True)
    p1 = 1.0 / (1.0 + jnp.exp(m2 - m1))
    p2 = 1.0 - p1
    idx_ref[...] = jnp.concatenate([a1, a2], axis=1)
    prob_ref[...] = jnp.concatenate([p1, p2], axis=1)


def _router(x_flat, gate_w):
    n, d = x_flat.shape
    tr = 1024
    return pl.pallas_call(
        _router_body,
        grid=(n // tr,),
        in_specs=[
            pl.BlockSpec((tr, d), lambda i: (i, 0)),
            pl.BlockSpec((E, d), lambda i: (0, 0)),
        ],
        out_specs=[
            pl.BlockSpec((tr, TOP_K), lambda i: (i, 0)),
            pl.BlockSpec((tr, TOP_K), lambda i: (i, 0)),
        ],
        out_shape=[
            jax.ShapeDtypeStruct((n, TOP_K), jnp.int32),
            jax.ShapeDtypeStruct((n, TOP_K), jnp.float32),
        ],
    )(x_flat, gate_w)


def _build_dispatch(sel_idx, sel_prob, n, np_total):
    e_flat = sel_idx.reshape(-1)
    oh = (e_flat[:, None] == jnp.arange(E, dtype=jnp.int32)[None, :]
          ).astype(jnp.int32)
    cum = jnp.cumsum(oh, axis=0)
    counts = cum[-1]
    rank = ((cum - oh) * oh).sum(axis=1)
    padded = ((counts + TM - 1) // TM) * TM
    ends = jnp.cumsum(padded)
    starts = ends - padded
    pos_flat = starts[e_flat] + rank
    tok = jnp.arange(n * TOP_K, dtype=jnp.int32) // TOP_K
    src_token = jnp.zeros((np_total,), jnp.int32).at[pos_flat].set(tok)
    w_sorted = jnp.zeros((np_total,), jnp.float32).at[pos_flat].set(
        sel_prob.reshape(-1))
    tile_starts = jnp.arange(np_total // TM, dtype=jnp.int32) * TM
    tile_expert = jnp.clip(
        jnp.searchsorted(ends, tile_starts, side="right"), 0, E - 1
    ).astype(jnp.int32)
    pos = pos_flat.reshape(n, TOP_K).astype(jnp.int32)
    return src_token, w_sorted, tile_expert, pos[:, 0], pos[:, 1]


def _sc_gather(x_i32, src_token, np_total):
    n, dw = x_i32.shape
    rows_per_w = np_total // NW
    chunk = 64
    mesh = plsc.VectorSubcoreMesh(core_axis_name="c", subcore_axis_name="s")

    @functools.partial(
        pl.kernel, mesh=mesh,
        out_type=jax.ShapeDtypeStruct((np_total, dw), jnp.int32),
        scratch_types=[
            pltpu.VMEM((chunk,), jnp.int32),
            pltpu.VMEM((chunk, dw), jnp.int32),
            pltpu.SemaphoreType.DMA,
        ],
    )
    def k(x_hbm, idx_hbm, out_hbm, idx_v, rows_v, sem):
        wid = lax.axis_index("s") * SC_CORES + lax.axis_index("c")
        base = wid * rows_per_w

        @pl.loop(0, rows_per_w, step=chunk)
        def _(off):
            pltpu.sync_copy(idx_hbm.at[pl.ds(base + off, chunk)], idx_v)
            pltpu.async_copy(x_hbm.at[idx_v], rows_v, sem).wait()
            pltpu.sync_copy(rows_v, out_hbm.at[pl.ds(base + off, chunk)])

    return k(x_i32, src_token)


def _gmm_body(te_ref, x_ref, w12_ref, w3_ref, ws_ref, o_ref):
    h = jax.lax.dot_general(
        x_ref[...], w12_ref[0], (((1,), (1,)), ((), ())),
        preferred_element_type=jnp.float32)
    hh = h.shape[1] // 2
    h1 = h[:, :hh]
    h2 = h[:, hh:]
    hidden = (h1 * jax.nn.sigmoid(h1) * h2).astype(jnp.bfloat16)
    o = jax.lax.dot_general(
        hidden, w3_ref[0], (((1,), (1,)), ((), ())),
        preferred_element_type=jnp.float32)
    o_ref[...] = o * ws_ref[0, 0][:, None]


def _gmm(x_sorted, fc12_bf, fc3_bf, w_sorted, tile_expert, np_total):
    d = x_sorted.shape[1]
    h2 = fc12_bf.shape[1]
    h = fc3_bf.shape[2]
    t_tiles = np_total // TM
    grid_spec = pltpu.PrefetchScalarGridSpec(
        num_scalar_prefetch=1,
        grid=(t_tiles,),
        in_specs=[
            pl.BlockSpec((TM, d), lambda i, te: (i, 0)),
            pl.BlockSpec((1, h2, d), lambda i, te: (te[i], 0, 0)),
            pl.BlockSpec((1, d, h), lambda i, te: (te[i], 0, 0)),
            pl.BlockSpec((1, 1, TM), lambda i, te: (i, 0, 0)),
        ],
        out_specs=pl.BlockSpec((TM, d), lambda i, te: (i, 0)),
    )
    return pl.pallas_call(
        _gmm_body,
        grid_spec=grid_spec,
        out_shape=jax.ShapeDtypeStruct((np_total, d), jnp.float32),
    )(tile_expert, x_sorted, fc12_bf, fc3_bf,
      w_sorted.reshape(t_tiles, 1, TM))


def _sc_combine(o_sorted, pos0, pos1):
    n = pos0.shape[0]
    d = o_sorted.shape[1]
    rows_per_w = n // NW
    chunk = 32
    lanes = 16
    mesh = plsc.VectorSubcoreMesh(core_axis_name="c", subcore_axis_name="s")

    @functools.partial(
        pl.kernel, mesh=mesh,
        out_type=jax.ShapeDtypeStruct((n, d), jnp.float32),
        scratch_types=[
            pltpu.VMEM((chunk,), jnp.int32),
            pltpu.VMEM((chunk,), jnp.int32),
            pltpu.VMEM((chunk, d), jnp.float32),
            pltpu.VMEM((chunk, d), jnp.float32),
            pltpu.SemaphoreType.DMA,
        ],
    )
    def k(o_hbm, p0_hbm, p1_hbm, out_hbm, i0_v, i1_v, g0_v, g1_v, sem):
        wid = lax.axis_index("s") * SC_CORES + lax.axis_index("c")
        base = wid * rows_per_w

        @pl.loop(0, rows_per_w, step=chunk)
        def _(off):
            pltpu.sync_copy(p0_hbm.at[pl.ds(base + off, chunk)], i0_v)
            pltpu.sync_copy(p1_hbm.at[pl.ds(base + off, chunk)], i1_v)
            pltpu.async_copy(o_hbm.at[i0_v], g0_v, sem).wait()
            pltpu.async_copy(o_hbm.at[i1_v], g1_v, sem).wait()

            @pl.loop(0, chunk)
            def _(r):
                @pl.loop(0, d, step=lanes)
                def _(c):
                    slc = (pl.ds(r, 1), pl.ds(c, lanes))
                    g0_v.at[*slc][...] = g0_v.at[*slc][...] + g1_v.at[*slc][...]

            pltpu.sync_copy(g0_v, out_hbm.at[pl.ds(base + off, chunk)])

    return k(o_sorted, pos0, pos1)


def kernel(x, gate_w, fc12_w, fc3_w):
    b, t, d = x.shape
    n = b * t
    np_total = n * TOP_K + E * TM

    x_flat = x.reshape(n, d)
    sel_idx, sel_prob = _router(x_flat, gate_w)
    src_token, w_sorted, tile_expert, pos0, pos1 = _build_dispatch(
        sel_idx, sel_prob, n, np_total)

    x_bf = x_flat.astype(jnp.bfloat16)
    x_i32 = jax.lax.bitcast_convert_type(
        x_bf.reshape(n, d // 2, 2), jnp.int32)
    xs_i32 = _sc_gather(x_i32, src_token, np_total)
    x_sorted = jax.lax.bitcast_convert_type(
        xs_i32, jnp.bfloat16).reshape(np_total, d)

    fc12_bf = fc12_w.astype(jnp.bfloat16)
    fc3_bf = fc3_w.astype(jnp.bfloat16)
    o_sorted = _gmm(x_sorted, fc12_bf, fc3_bf, w_sorted, tile_expert, np_total)

    out_flat = _sc_combine(o_sorted, pos0, pos1)
    return out_flat.reshape(b, t, d)


# speedup vs baseline: 1.4805x; 1.4805x over previous
import functools

import jax
import jax.numpy as jnp
from jax import lax
from jax.experimental import pallas as pl
from jax.experimental.pallas import tpu as pltpu
from jax.experimental.pallas import tpu_sc as plsc

E = 8
TOP_K = 2
TM = 256
NEG_INF = -1e30

SC_CORES = 2
SC_SUBCORES = 16
NW = SC_CORES * SC_SUBCORES


def _router_body(x_ref, gw_ref, idx_ref, prob_ref):
    scores = jax.lax.dot_general(
        x_ref[...], gw_ref[...], (((1,), (1,)), ((), ())),
        preferred_element_type=jnp.float32,
        precision=jax.lax.Precision.DEFAULT)
    tr = scores.shape[0]
    iota = jax.lax.broadcasted_iota(jnp.int32, (tr, E), 1)
    m1 = jnp.max(scores, axis=1, keepdims=True)
    a1 = jnp.min(jnp.where(scores == m1, iota, E), axis=1, keepdims=True)
    masked = jnp.where(iota == a1, NEG_INF, scores)
    m2 = jnp.max(masked, axis=1, keepdims=True)
    a2 = jnp.min(jnp.where(masked == m2, iota, E), axis=1, keepdims=True)
    p1 = 1.0 / (1.0 + jnp.exp(m2 - m1))
    p2 = 1.0 - p1
    idx_ref[...] = jnp.concatenate([a1, a2], axis=1)
    prob_ref[...] = jnp.concatenate([p1, p2], axis=1)


def _router(x_flat, gate_w):
    n, d = x_flat.shape
    tr = 1024
    return pl.pallas_call(
        _router_body,
        grid=(n // tr,),
        in_specs=[
            pl.BlockSpec((tr, d), lambda i: (i, 0)),
            pl.BlockSpec((E, d), lambda i: (0, 0)),
        ],
        out_specs=[
            pl.BlockSpec((tr, TOP_K), lambda i: (i, 0)),
            pl.BlockSpec((tr, TOP_K), lambda i: (i, 0)),
        ],
        out_shape=[
            jax.ShapeDtypeStruct((n, TOP_K), jnp.int32),
            jax.ShapeDtypeStruct((n, TOP_K), jnp.float32),
        ],
    )(x_flat, gate_w)


def _build_dispatch(sel_idx, sel_prob, n, np_total):
    e_flat = sel_idx.reshape(-1)
    oh = (e_flat[:, None] == jnp.arange(E, dtype=jnp.int32)[None, :]
          ).astype(jnp.int32)
    cum = jnp.cumsum(oh, axis=0)
    counts = cum[-1]
    rank = ((cum - oh) * oh).sum(axis=1)
    padded = ((counts + TM - 1) // TM) * TM
    ends = jnp.cumsum(padded)
    starts = ends - padded
    pos_flat = starts[e_flat] + rank
    tok = jnp.arange(n * TOP_K, dtype=jnp.int32) // TOP_K
    src_token = jnp.zeros((np_total,), jnp.int32).at[pos_flat].set(tok)
    w_sorted = jnp.zeros((np_total,), jnp.float32).at[pos_flat].set(
        sel_prob.reshape(-1))
    tile_starts = jnp.arange(np_total // TM, dtype=jnp.int32) * TM
    tile_expert = jnp.clip(
        jnp.searchsorted(ends, tile_starts, side="right"), 0, E - 1
    ).astype(jnp.int32)
    pos = pos_flat.reshape(n, TOP_K).astype(jnp.int32)
    return src_token, w_sorted, tile_expert, pos[:, 0], pos[:, 1]


def _sc_gather(x_flat, src_token, np_total):
    n, d = x_flat.shape
    rows_per_w = np_total // NW
    chunk = 64
    mesh = plsc.VectorSubcoreMesh(core_axis_name="c", subcore_axis_name="s")

    @functools.partial(
        pl.kernel, mesh=mesh,
        out_type=jax.ShapeDtypeStruct((np_total, d), x_flat.dtype),
        scratch_types=[
            pltpu.VMEM((chunk,), jnp.int32),
            pltpu.VMEM((chunk, d), x_flat.dtype),
            pltpu.SemaphoreType.DMA,
        ],
    )
    def k(x_hbm, idx_hbm, out_hbm, idx_v, rows_v, sem):
        wid = lax.axis_index("s") * SC_CORES + lax.axis_index("c")
        base = wid * rows_per_w

        @pl.loop(0, rows_per_w, step=chunk)
        def _(off):
            pltpu.sync_copy(idx_hbm.at[pl.ds(base + off, chunk)], idx_v)
            pltpu.async_copy(x_hbm.at[idx_v], rows_v, sem).wait()
            pltpu.sync_copy(rows_v, out_hbm.at[pl.ds(base + off, chunk)])

    return k(x_flat, src_token)


def _gmm_body(te_ref, x_ref, w12h_ref, w3_ref, ws_ref, o_ref, h_ref):
    j = pl.program_id(1)
    hh = w12h_ref.shape[1]
    hpart = jax.lax.dot_general(
        x_ref[...], w12h_ref[0], (((1,), (1,)), ((), ())),
        preferred_element_type=jnp.float32)
    h_ref[:, pl.ds(j * hh, hh)] = hpart

    @pl.when(j == 1)
    def _():
        h1 = h_ref[:, :hh]
        h2 = h_ref[:, hh:]
        hidden = h1 * jax.nn.sigmoid(h1) * h2
        o = jax.lax.dot_general(
            hidden, w3_ref[0], (((1,), (1,)), ((), ())),
            preferred_element_type=jnp.float32)
        o_ref[...] = o * ws_ref[0, 0][:, None]


def _gmm(x_sorted, fc12_w, fc3_w, w_sorted, tile_expert, np_total):
    d = x_sorted.shape[1]
    hh = fc12_w.shape[1] // 2
    h = fc3_w.shape[2]
    t_tiles = np_total // TM
    grid_spec = pltpu.PrefetchScalarGridSpec(
        num_scalar_prefetch=1,
        grid=(t_tiles, 2),
        in_specs=[
            pl.BlockSpec((TM, d), lambda i, j, te: (i, 0)),
            pl.BlockSpec((1, hh, d), lambda i, j, te: (te[i], j, 0)),
            pl.BlockSpec((1, d, h), lambda i, j, te: (te[i], 0, 0)),
            pl.BlockSpec((1, 1, TM), lambda i, j, te: (i, 0, 0)),
        ],
        out_specs=pl.BlockSpec((TM, d), lambda i, j, te: (i, 0)),
        scratch_shapes=[pltpu.VMEM((TM, 2 * hh), jnp.float32)],
    )
    return pl.pallas_call(
        _gmm_body,
        grid_spec=grid_spec,
        out_shape=jax.ShapeDtypeStruct((np_total, d), jnp.float32),
    )(tile_expert, x_sorted, fc12_w, fc3_w,
      w_sorted.reshape(t_tiles, 1, TM))


def _sc_combine(o_sorted, pos0, pos1):
    n = pos0.shape[0]
    d = o_sorted.shape[1]
    rows_per_w = n // NW
    chunk = 32
    lanes = 16
    mesh = plsc.VectorSubcoreMesh(core_axis_name="c", subcore_axis_name="s")

    @functools.partial(
        pl.kernel, mesh=mesh,
        out_type=jax.ShapeDtypeStruct((n, d), jnp.float32),
        scratch_types=[
            pltpu.VMEM((chunk,), jnp.int32),
            pltpu.VMEM((chunk,), jnp.int32),
            pltpu.VMEM((chunk, d), jnp.float32),
            pltpu.VMEM((chunk, d), jnp.float32),
            pltpu.SemaphoreType.DMA,
        ],
    )
    def k(o_hbm, p0_hbm, p1_hbm, out_hbm, i0_v, i1_v, g0_v, g1_v, sem):
        wid = lax.axis_index("s") * SC_CORES + lax.axis_index("c")
        base = wid * rows_per_w

        @pl.loop(0, rows_per_w, step=chunk)
        def _(off):
            pltpu.sync_copy(p0_hbm.at[pl.ds(base + off, chunk)], i0_v)
            pltpu.sync_copy(p1_hbm.at[pl.ds(base + off, chunk)], i1_v)
            pltpu.async_copy(o_hbm.at[i0_v], g0_v, sem).wait()
            pltpu.async_copy(o_hbm.at[i1_v], g1_v, sem).wait()

            @pl.loop(0, chunk)
            def _(r):
                @pl.loop(0, d, step=lanes)
                def _(c):
                    slc = (pl.ds(r, 1), pl.ds(c, lanes))
                    g0_v.at[*slc][...] = g0_v.at[*slc][...] + g1_v.at[*slc][...]

            pltpu.sync_copy(g0_v, out_hbm.at[pl.ds(base + off, chunk)])

    return k(o_sorted, pos0, pos1)


def kernel(x, gate_w, fc12_w, fc3_w):
    b, t, d = x.shape
    n = b * t
    np_total = n * TOP_K + E * TM

    x_flat = x.reshape(n, d)
    sel_idx, sel_prob = _router(x_flat, gate_w)
    src_token, w_sorted, tile_expert, pos0, pos1 = _build_dispatch(
        sel_idx, sel_prob, n, np_total)

    x_sorted = _sc_gather(x_flat, src_token, np_total)
    o_sorted = _gmm(x_sorted, fc12_w, fc3_w, w_sorted, tile_expert, np_total)

    out_flat = _sc_combine(o_sorted, pos0, pos1)
    return out_flat.reshape(b, t, d)


# speedup vs baseline: 1.9088x; 1.2893x over previous
import functools

import jax
import jax.numpy as jnp
from jax import lax
from jax.experimental import pallas as pl
from jax.experimental.pallas import tpu as pltpu
from jax.experimental.pallas import tpu_sc as plsc

E = 8
TOP_K = 2
TM = 256
NEG_INF = -1e30

SC_CORES = 2
SC_SUBCORES = 16
NW = SC_CORES * SC_SUBCORES


def _router_body(x_ref, gw_ref, idx_ref, prob_ref):
    scores = jax.lax.dot_general(
        x_ref[...], gw_ref[...], (((1,), (1,)), ((), ())),
        preferred_element_type=jnp.float32,
        precision=jax.lax.Precision.DEFAULT)
    tr = scores.shape[0]
    iota = jax.lax.broadcasted_iota(jnp.int32, (tr, E), 1)
    m1 = jnp.max(scores, axis=1, keepdims=True)
    a1 = jnp.min(jnp.where(scores == m1, iota, E), axis=1, keepdims=True)
    masked = jnp.where(iota == a1, NEG_INF, scores)
    m2 = jnp.max(masked, axis=1, keepdims=True)
    a2 = jnp.min(jnp.where(masked == m2, iota, E), axis=1, keepdims=True)
    p1 = 1.0 / (1.0 + jnp.exp(m2 - m1))
    p2 = 1.0 - p1
    idx_ref[...] = jnp.concatenate([a1, a2], axis=1)
    prob_ref[...] = jnp.concatenate([p1, p2], axis=1)


def _router(x_flat, gate_w):
    n, d = x_flat.shape
    tr = 1024
    return pl.pallas_call(
        _router_body,
        grid=(n // tr,),
        in_specs=[
            pl.BlockSpec((tr, d), lambda i: (i, 0)),
            pl.BlockSpec((E, d), lambda i: (0, 0)),
        ],
        out_specs=[
            pl.BlockSpec((tr, TOP_K), lambda i: (i, 0)),
            pl.BlockSpec((tr, TOP_K), lambda i: (i, 0)),
        ],
        out_shape=[
            jax.ShapeDtypeStruct((n, TOP_K), jnp.int32),
            jax.ShapeDtypeStruct((n, TOP_K), jnp.float32),
        ],
    )(x_flat, gate_w)


def _build_dispatch(sel_idx, sel_prob, n, np_total):
    e_flat = sel_idx.reshape(-1)
    oh = (e_flat[:, None] == jnp.arange(E, dtype=jnp.int32)[None, :]
          ).astype(jnp.int32)
    cum = jnp.cumsum(oh, axis=0)
    counts = cum[-1]
    rank = ((cum - oh) * oh).sum(axis=1)
    padded = ((counts + TM - 1) // TM) * TM
    ends = jnp.cumsum(padded)
    starts = ends - padded
    pos_flat = starts[e_flat] + rank
    tok = jnp.arange(n * TOP_K, dtype=jnp.int32) // TOP_K
    src_token = jnp.zeros((np_total,), jnp.int32).at[pos_flat].set(tok)
    w_sorted = jnp.zeros((np_total,), jnp.float32).at[pos_flat].set(
        sel_prob.reshape(-1))
    tile_starts = jnp.arange(np_total // TM, dtype=jnp.int32) * TM
    tile_expert = jnp.clip(
        jnp.searchsorted(ends, tile_starts, side="right"), 0, E - 1
    ).astype(jnp.int32)
    return src_token, w_sorted, tile_expert, pos_flat.astype(jnp.int32)


def _sc_gather(x_flat, src_token, np_total):
    n, d = x_flat.shape
    rows_per_w = np_total // NW
    chunk = 48
    nchunks = rows_per_w // chunk
    assert nchunks * chunk == rows_per_w and nchunks % 2 == 0
    mesh = plsc.VectorSubcoreMesh(core_axis_name="c", subcore_axis_name="s")

    @functools.partial(
        pl.kernel, mesh=mesh,
        out_type=jax.ShapeDtypeStruct((np_total, d), x_flat.dtype),
        scratch_types=[
            pltpu.VMEM((rows_per_w,), jnp.int32),
            pltpu.VMEM((chunk, d), x_flat.dtype),
            pltpu.VMEM((chunk, d), x_flat.dtype),
            pltpu.SemaphoreType.DMA,
            pltpu.SemaphoreType.DMA,
        ],
    )
    def k(x_hbm, idx_hbm, out_hbm, idx_v, r0, r1, s0, s1):
        wid = lax.axis_index("s") * SC_CORES + lax.axis_index("c")
        base = wid * rows_per_w
        pltpu.sync_copy(idx_hbm.at[pl.ds(base, rows_per_w)], idx_v)

        def issue(buf, sem, c):
            pltpu.async_copy(x_hbm.at[idx_v.at[pl.ds(c * chunk, chunk)]],
                             buf, sem)

        def wait(buf, sem):
            pltpu.make_async_copy(
                x_hbm.at[idx_v.at[pl.ds(0, chunk)]], buf, sem).wait()

        issue(r0, s0, 0)

        @pl.loop(0, nchunks, step=2)
        def _(i):
            issue(r1, s1, i + 1)
            wait(r0, s0)
            pltpu.sync_copy(r0, out_hbm.at[pl.ds(base + i * chunk, chunk)])

            @pl.when(i + 2 < nchunks)
            def _():
                issue(r0, s0, i + 2)

            wait(r1, s1)
            pltpu.sync_copy(
                r1, out_hbm.at[pl.ds(base + (i + 1) * chunk, chunk)])

    return k(x_flat, src_token)


def _gmm_body(te_ref, x_ref, w12_ref, w3_ref, ws_ref, o_ref):
    h = jax.lax.dot_general(
        x_ref[...], w12_ref[0], (((1,), (1,)), ((), ())),
        preferred_element_type=jnp.float32)
    hh = h.shape[1] // 2
    h1 = h[:, :hh]
    h2 = h[:, hh:]
    hidden = h1 * jax.nn.sigmoid(h1) * h2
    o = jax.lax.dot_general(
        hidden, w3_ref[0], (((1,), (1,)), ((), ())),
        preferred_element_type=jnp.float32)
    o_ref[...] = o * ws_ref[0, 0][:, None]


def _gmm(x_sorted, fc12_w, fc3_w, w_sorted, tile_expert, np_total):
    d = x_sorted.shape[1]
    h2 = fc12_w.shape[1]
    h = fc3_w.shape[2]
    t_tiles = np_total // TM
    grid_spec = pltpu.PrefetchScalarGridSpec(
        num_scalar_prefetch=1,
        grid=(t_tiles,),
        in_specs=[
            pl.BlockSpec((TM, d), lambda i, te: (i, 0)),
            pl.BlockSpec((1, h2, d), lambda i, te: (te[i], 0, 0)),
            pl.BlockSpec((1, d, h), lambda i, te: (te[i], 0, 0)),
            pl.BlockSpec((1, 1, TM), lambda i, te: (i, 0, 0)),
        ],
        out_specs=pl.BlockSpec((TM, d), lambda i, te: (i, 0)),
    )
    return pl.pallas_call(
        _gmm_body,
        grid_spec=grid_spec,
        out_shape=jax.ShapeDtypeStruct((np_total, d), jnp.float32),
    )(tile_expert, x_sorted, fc12_w, fc3_w,
      w_sorted.reshape(t_tiles, 1, TM))


def _sc_combine(o_sorted, pos_flat, n):
    d = o_sorted.shape[1]
    rows_per_w = n // NW
    ct = 16
    nchunks = rows_per_w // ct
    assert nchunks * ct == rows_per_w and nchunks % 2 == 0
    lanes = 16
    mesh = plsc.VectorSubcoreMesh(core_axis_name="c", subcore_axis_name="s")

    @functools.partial(
        pl.kernel, mesh=mesh,
        out_type=jax.ShapeDtypeStruct((n, d), jnp.float32),
        scratch_types=[
            pltpu.VMEM((2 * rows_per_w,), jnp.int32),
            pltpu.VMEM((2 * ct, d), jnp.float32),
            pltpu.VMEM((2 * ct, d), jnp.float32),
            pltpu.VMEM((ct, d), jnp.float32),
            pltpu.SemaphoreType.DMA,
            pltpu.SemaphoreType.DMA,
        ],
    )
    def k(o_hbm, pf_hbm, out_hbm, idx_v, g0, g1, ob, s0, s1):
        wid = lax.axis_index("s") * SC_CORES + lax.axis_index("c")
        base = wid * rows_per_w
        pltpu.sync_copy(pf_hbm.at[pl.ds(2 * base, 2 * rows_per_w)], idx_v)

        def issue(buf, sem, c):
            pltpu.async_copy(
                o_hbm.at[idx_v.at[pl.ds(c * 2 * ct, 2 * ct)]], buf, sem)

        def wait(buf, sem):
            pltpu.make_async_copy(
                o_hbm.at[idx_v.at[pl.ds(0, 2 * ct)]], buf, sem).wait()

        def add_store(buf, c):
            @pl.loop(0, ct)
            def _(r):
                @pl.loop(0, d, step=lanes)
                def _(col):
                    cs = pl.ds(col, lanes)
                    ob.at[pl.ds(r, 1), cs][...] = (
                        buf.at[pl.ds(2 * r, 1), cs][...]
                        + buf.at[pl.ds(2 * r + 1, 1), cs][...])

            pltpu.sync_copy(ob, out_hbm.at[pl.ds(base + c * ct, ct)])

        issue(g0, s0, 0)

        @pl.loop(0, nchunks, step=2)
        def _(i):
            issue(g1, s1, i + 1)
            wait(g0, s0)
            add_store(g0, i)

            @pl.when(i + 2 < nchunks)
            def _():
                issue(g0, s0, i + 2)

            wait(g1, s1)
            add_store(g1, i + 1)

    return k(o_sorted, pos_flat)


def kernel(x, gate_w, fc12_w, fc3_w):
    b, t, d = x.shape
    n = b * t
    np_total = n * TOP_K + E * TM

    x_flat = x.reshape(n, d)
    sel_idx, sel_prob = _router(x_flat, gate_w)
    src_token, w_sorted, tile_expert, pos_flat = _build_dispatch(
        sel_idx, sel_prob, n, np_total)

    x_sorted = _sc_gather(x_flat, src_token, np_total)
    o_sorted = _gmm(x_sorted, fc12_w, fc3_w, w_sorted, tile_expert, np_total)

    out_flat = _sc_combine(o_sorted, pos_flat, n)
    return out_flat.reshape(b, t, d)
